# baseline (speedup 1.0000x reference)
import functools

import jax
import jax.numpy as jnp
from jax import lax
from jax.experimental import pallas as pl
from jax.experimental.pallas import tpu as pltpu
from jax.experimental.pallas import tpu_sc as plsc

D_MODEL = 1024
D4 = D_MODEL // 4
B = 4
S = 4096
NC = 2
NS = 16
NW = NC * NS
S_SC = 1536
S_TC = S - S_SC
S_PER_W = S_SC // NW
CHUNK = 4
N_CHUNKS = S_PER_W // CHUNK
NSETS = 4
L = 16
TC_BLK = 512
K_TAB = 64


def _sc_kernel(x_hbm, ts_hbm, vol_hbm, pe_hbm, h_hbm, d_hbm, m_hbm, wb_hbm,
               consts_hbm, out_hbm, *scratch):
    xs = [list(scratch[s * B:(s + 1) * B]) for s in range(NSETS)]
    pebufs = list(scratch[16:20])
    htab, dtab, mtab, wb = scratch[20:24]
    tsv, volv, hbv, dbv, mbv, cbuf = scratch[24:30]
    lsems = list(scratch[30:34])
    ssems = list(scratch[34:38])

    wid = lax.axis_index("s") * NC + lax.axis_index("c")
    s0 = S_TC + wid * S_PER_W
    o0 = wid * S_PER_W

    pltpu.sync_copy(h_hbm, htab)
    pltpu.sync_copy(d_hbm, dtab)
    pltpu.sync_copy(m_hbm, mtab)
    pltpu.sync_copy(wb_hbm, wb)
    pltpu.sync_copy(consts_hbm, cbuf)
    for b in range(B):
        pltpu.sync_copy(ts_hbm.at[pl.ds(b * S + s0, S_PER_W)], tsv.at[b])
        pltpu.sync_copy(vol_hbm.at[pl.ds(b * S + s0, S_PER_W)], volv.at[b])

    def idx_body(g, carry):
        off = g * L
        sl = pl.ds(off, L)
        for b in range(B):
            tsg = tsv[b, sl].astype(jnp.uint32)
            cu = [cbuf[i, :].astype(jnp.uint32) for i in range(5)]
            q1 = lax.div(tsg, cu[0])
            days = lax.div(q1, cu[1])
            hbv[b, sl] = ((q1 - days * cu[1]).astype(jnp.int32)) << 8
            dbv[b, sl] = ((days - lax.div(days, cu[2]) * cu[2])
                          .astype(jnp.int32)) << 8
            mo = lax.div(days, cu[3])
            mbv[b, sl] = ((mo - lax.div(mo, cu[4]) * cu[4])
                          .astype(jnp.int32)) << 8
        return carry

    lax.fori_loop(0, S_PER_W // L, idx_body, 0, unroll=False)

    iot = lax.broadcasted_iota(jnp.int32, (L,), 0)

    def start_load(s, row0):
        pltpu.async_copy(pe_hbm.at[pl.ds(s0 + row0, CHUNK), :],
                         pebufs[s], lsems[s])
        for b in range(B):
            pltpu.async_copy(x_hbm.at[b, pl.ds(s0 + row0, CHUNK), :],
                             xs[s][b], lsems[s])

    def wait_load(s):
        pltpu.make_async_copy(pe_hbm.at[pl.ds(0, CHUNK), :],
                              pebufs[s], lsems[s]).wait()
        for b in range(B):
            pltpu.make_async_copy(x_hbm.at[b, pl.ds(0, CHUNK), :],
                                  xs[s][b], lsems[s]).wait()

    def start_store(s, row0):
        for b in range(B):
            pltpu.async_copy(xs[s][b],
                             out_hbm.at[b, pl.ds(o0 + row0, CHUNK), :],
                             ssems[s])

    def wait_store(s):
        for b in range(B):
            pltpu.make_async_copy(xs[s][b],
                                  out_hbm.at[b, pl.ds(0, CHUNK), :],
                                  ssems[s]).wait()

    def compute(s, c):
        pebuf = pebufs[s]
        xbufs = xs[s]

        def token_body(t, carry):
            col = c * CHUNK + t
            hv = [None] * B
            dv = [None] * B
            mv = [None] * B
            vv = [None] * B
            tvec = jnp.full((L,), t, dtype=jnp.int32)
            cvec = jnp.full((L,), col, dtype=jnp.int32)
            for b in range(B):
                bvec = jnp.full((L,), b, dtype=jnp.int32)
                hv[b] = plsc.load_gather(hbv, [bvec, cvec])
                dv[b] = plsc.load_gather(dbv, [bvec, cvec])
                mv[b] = plsc.load_gather(mbv, [bvec, cvec])
                vv[b] = plsc.load_gather(volv, [bvec, cvec])
            for j in range(16):
                colj = iot + (j * L)
                col1 = colj + D4
                col2 = colj + 2 * D4
                col3 = colj + 3 * D4
                c0 = pl.ds(j * L, L)
                c1 = pl.ds(D4 + j * L, L)
                c2 = pl.ds(2 * D4 + j * L, L)
                c3 = pl.ds(3 * D4 + j * L, L)
                pe0 = pebuf[t, c0]
                pe1 = pebuf[t, c1]
                pe2 = pebuf[t, c2]
                pe3 = pebuf[t, c3]
                wW = wb[0, c0]
                wB = wb[1, c0]
                for b in range(B):
                    xb = xbufs[b]
                    hrow = plsc.load_gather(htab, [hv[b] + colj])
                    drow = plsc.load_gather(dtab, [dv[b] + colj])
                    mrow = plsc.load_gather(mtab, [mv[b] + colj])
                    plsc.addupdate_scatter(xb, [tvec, colj], pe0 + hrow)
                    plsc.addupdate_scatter(xb, [tvec, col1], pe1 + drow)
                    plsc.addupdate_scatter(xb, [tvec, col2], pe2 + mrow)
                    plsc.addupdate_scatter(xb, [tvec, col3],
                                           pe3 + (wW * vv[b] + wB))
            return carry

        lax.fori_loop(0, CHUNK, token_body, 0, unroll=False)

    start_load(0, 0)

    def pipe_body(k, carry):
        for p in range(NSETS):
            i = k * NSETS + p
            s_next = (p + 1) % NSETS

            @pl.when(i >= NSETS - 1)
            def _():
                wait_store(s_next)

            @pl.when(i + 1 < N_CHUNKS)
            def _():
                start_load(s_next, (i + 1) * CHUNK)

            wait_load(p)
            compute(p, i)
            start_store(p, i * CHUNK)
        return carry

    lax.fori_loop(0, N_CHUNKS // NSETS, pipe_body, 0, unroll=False)

    for s in range(1, NSETS):
        wait_store(s)


def _tc_kernel(x_ref, ts_ref, vol_ref, pe_ref, ct_ref, out_ref):
    j = pl.program_id(0)
    ts = ts_ref[0, j, :].reshape(1, TC_BLK)
    hh = (ts // 3600) % 24
    dd = (ts // 86400) % 7
    mm = (ts // 2592000) % 12
    k = lax.broadcasted_iota(jnp.int32, (K_TAB, TC_BLK), 0)
    onehot = ((k == hh) | (k == (24 + dd)) | (k == (31 + mm))
              | (k == 43)).astype(jnp.float32)
    vol = vol_ref[0, j, :].reshape(1, TC_BLK)
    at = jnp.where(k == 44, vol, onehot)
    enc = lax.dot_general(at, ct_ref[...], (((0,), (0,)), ((), ())),
                          preferred_element_type=jnp.float32)
    out_ref[0, :, :] = x_ref[0, :, :] + pe_ref[...] + enc


def _run_tc(x, ts, vol2d, pe_s, ctable):
    grid = (S_TC // TC_BLK, B)
    ts3 = ts.reshape(B, S // TC_BLK, TC_BLK)
    vol3 = vol2d.reshape(B, S // TC_BLK, TC_BLK)
    return pl.pallas_call(
        _tc_kernel,
        grid=grid,
        in_specs=[
            pl.BlockSpec((1, TC_BLK, D_MODEL), lambda j, b: (b, j, 0)),
            pl.BlockSpec((1, S // TC_BLK, TC_BLK), lambda j, b: (b, 0, 0)),
            pl.BlockSpec((1, S // TC_BLK, TC_BLK), lambda j, b: (b, 0, 0)),
            pl.BlockSpec((TC_BLK, D_MODEL), lambda j, b: (j, 0)),
            pl.BlockSpec((K_TAB, D_MODEL), lambda j, b: (0, 0)),
        ],
        out_specs=pl.BlockSpec((1, TC_BLK, D_MODEL), lambda j, b: (b, j, 0)),
        out_shape=jax.ShapeDtypeStruct((B, S, D_MODEL), jnp.float32),
    )(x, ts3, vol3, pe_s, ctable)


def kernel(x, timestamps, volatility_regime, pe, hour_table, day_table,
           month_table, regime_W, regime_b):
    vol2d = volatility_regime[..., 0]
    wb = jnp.stack([regime_W[:, 0], regime_b])
    pe_s = pe
    consts = jnp.tile(
        jnp.array([3600, 24, 7, 30, 12], jnp.int32)[:, None], (1, L))
    ct = jnp.zeros((K_TAB, D_MODEL), jnp.float32)
    ct = ct.at[0:24, 0:D4].set(hour_table)
    ct = ct.at[24:31, D4:2 * D4].set(day_table)
    ct = ct.at[31:43, 2 * D4:3 * D4].set(month_table)
    ct = ct.at[43, 3 * D4:].set(regime_b)
    ct = ct.at[44, 3 * D4:].set(regime_W[:, 0])

    mesh = plsc.VectorSubcoreMesh(core_axis_name="c", subcore_axis_name="s")
    scratch = []
    for _ in range(NSETS * B):
        scratch.append(pltpu.VMEM((CHUNK, D_MODEL), jnp.float32))
    for _ in range(NSETS):
        scratch.append(pltpu.VMEM((CHUNK, D_MODEL), jnp.float32))
    scratch += [
        pltpu.VMEM((24 * D4,), jnp.float32),
        pltpu.VMEM((7 * D4,), jnp.float32),
        pltpu.VMEM((12 * D4,), jnp.float32),
        pltpu.VMEM((2, D4), jnp.float32),
        pltpu.VMEM((B, S_PER_W), jnp.int32),
        pltpu.VMEM((B, S_PER_W), jnp.float32),
        pltpu.VMEM((B, S_PER_W), jnp.int32),
        pltpu.VMEM((B, S_PER_W), jnp.int32),
        pltpu.VMEM((B, S_PER_W), jnp.int32),
        pltpu.VMEM((5, L), jnp.int32),
    ]
    scratch += [pltpu.SemaphoreType.DMA] * (2 * NSETS)

    run = functools.partial(
        pl.kernel,
        out_type=jax.ShapeDtypeStruct((B, S_SC, D_MODEL), jnp.float32),
        mesh=mesh,
        compiler_params=pltpu.CompilerParams(needs_layout_passes=False),
        scratch_types=scratch,
    )(_sc_kernel)
    sc_part = run(x, timestamps.reshape(-1), vol2d.reshape(-1), pe_s,
                  hour_table.reshape(-1), day_table.reshape(-1),
                  month_table.reshape(-1), wb, consts)
    tc_out = _run_tc(x, timestamps, vol2d, pe_s, ct)
    return lax.dynamic_update_slice(tc_out, sc_part, (0, S_TC, 0))


# speedup vs baseline: 1.0998x; 1.0998x over previous
import functools

import jax
import jax.numpy as jnp
from jax import lax
from jax.experimental import pallas as pl
from jax.experimental.pallas import tpu as pltpu
from jax.experimental.pallas import tpu_sc as plsc

D_MODEL = 1024
D4 = D_MODEL // 4
B = 4
S = 4096
NC = 2
NS = 16
NW = NC * NS
S_SC = 512
S_TC = S - S_SC
S_PER_W = S_SC // NW
CHUNK = 4
N_CHUNKS = S_PER_W // CHUNK
NSETS = 4
L = 16
TC_BLK = 512
K_TAB = 64


def _sc_kernel(x_hbm, ts_hbm, vol_hbm, pe_hbm, h_hbm, d_hbm, m_hbm, wb_hbm,
               consts_hbm, out_hbm, *scratch):
    xs = [list(scratch[s * B:(s + 1) * B]) for s in range(NSETS)]
    pebufs = list(scratch[16:20])
    htab, dtab, mtab, wb = scratch[20:24]
    tsv, volv, hbv, dbv, mbv, cbuf = scratch[24:30]
    lsems = list(scratch[30:34])
    ssems = list(scratch[34:38])

    wid = lax.axis_index("s") * NC + lax.axis_index("c")
    s0 = S_TC + wid * S_PER_W
    o0 = wid * S_PER_W

    pltpu.sync_copy(h_hbm, htab)
    pltpu.sync_copy(d_hbm, dtab)
    pltpu.sync_copy(m_hbm, mtab)
    pltpu.sync_copy(wb_hbm, wb)
    pltpu.sync_copy(consts_hbm, cbuf)
    for b in range(B):
        pltpu.sync_copy(ts_hbm.at[pl.ds(b * S + s0, S_PER_W)], tsv.at[b])
        pltpu.sync_copy(vol_hbm.at[pl.ds(b * S + s0, S_PER_W)], volv.at[b])

    def idx_body(g, carry):
        off = g * L
        sl = pl.ds(off, L)
        for b in range(B):
            tsg = tsv[b, sl].astype(jnp.uint32)
            cu = [cbuf[i, :].astype(jnp.uint32) for i in range(5)]
            q1 = lax.div(tsg, cu[0])
            days = lax.div(q1, cu[1])
            hbv[b, sl] = ((q1 - days * cu[1]).astype(jnp.int32)) << 8
            dbv[b, sl] = ((days - lax.div(days, cu[2]) * cu[2])
                          .astype(jnp.int32)) << 8
            mo = lax.div(days, cu[3])
            mbv[b, sl] = ((mo - lax.div(mo, cu[4]) * cu[4])
                          .astype(jnp.int32)) << 8
        return carry

    lax.fori_loop(0, S_PER_W // L, idx_body, 0, unroll=False)

    iot = lax.broadcasted_iota(jnp.int32, (L,), 0)

    def start_load(s, row0):
        pltpu.async_copy(pe_hbm.at[pl.ds(s0 + row0, CHUNK), :],
                         pebufs[s], lsems[s])
        for b in range(B):
            pltpu.async_copy(x_hbm.at[b, pl.ds(s0 + row0, CHUNK), :],
                             xs[s][b], lsems[s])

    def wait_load(s):
        pltpu.make_async_copy(pe_hbm.at[pl.ds(0, CHUNK), :],
                              pebufs[s], lsems[s]).wait()
        for b in range(B):
            pltpu.make_async_copy(x_hbm.at[b, pl.ds(0, CHUNK), :],
                                  xs[s][b], lsems[s]).wait()

    def start_store(s, row0):
        for b in range(B):
            pltpu.async_copy(xs[s][b],
                             out_hbm.at[b, pl.ds(o0 + row0, CHUNK), :],
                             ssems[s])

    def wait_store(s):
        for b in range(B):
            pltpu.make_async_copy(xs[s][b],
                                  out_hbm.at[b, pl.ds(0, CHUNK), :],
                                  ssems[s]).wait()

    def compute(s, c):
        pebuf = pebufs[s]
        xbufs = xs[s]

        def token_body(t, carry):
            col = c * CHUNK + t
            hv = [None] * B
            dv = [None] * B
            mv = [None] * B
            vv = [None] * B
            tvec = jnp.full((L,), t, dtype=jnp.int32)
            cvec = jnp.full((L,), col, dtype=jnp.int32)
            for b in range(B):
                bvec = jnp.full((L,), b, dtype=jnp.int32)
                hv[b] = plsc.load_gather(hbv, [bvec, cvec])
                dv[b] = plsc.load_gather(dbv, [bvec, cvec])
                mv[b] = plsc.load_gather(mbv, [bvec, cvec])
                vv[b] = plsc.load_gather(volv, [bvec, cvec])
            for j in range(16):
                colj = iot + (j * L)
                col1 = colj + D4
                col2 = colj + 2 * D4
                col3 = colj + 3 * D4
                c0 = pl.ds(j * L, L)
                c1 = pl.ds(D4 + j * L, L)
                c2 = pl.ds(2 * D4 + j * L, L)
                c3 = pl.ds(3 * D4 + j * L, L)
                pe0 = pebuf[t, c0]
                pe1 = pebuf[t, c1]
                pe2 = pebuf[t, c2]
                pe3 = pebuf[t, c3]
                wW = wb[0, c0]
                wB = wb[1, c0]
                for b in range(B):
                    xb = xbufs[b]
                    hrow = plsc.load_gather(htab, [hv[b] + colj])
                    drow = plsc.load_gather(dtab, [dv[b] + colj])
                    mrow = plsc.load_gather(mtab, [mv[b] + colj])
                    plsc.addupdate_scatter(xb, [tvec, colj], pe0 + hrow)
                    plsc.addupdate_scatter(xb, [tvec, col1], pe1 + drow)
                    plsc.addupdate_scatter(xb, [tvec, col2], pe2 + mrow)
                    plsc.addupdate_scatter(xb, [tvec, col3],
                                           pe3 + (wW * vv[b] + wB))
            return carry

        lax.fori_loop(0, CHUNK, token_body, 0, unroll=False)

    start_load(0, 0)

    def pipe_body(k, carry):
        for p in range(NSETS):
            i = k * NSETS + p
            s_next = (p + 1) % NSETS

            @pl.when(i >= NSETS - 1)
            def _():
                wait_store(s_next)

            @pl.when(i + 1 < N_CHUNKS)
            def _():
                start_load(s_next, (i + 1) * CHUNK)

            wait_load(p)
            compute(p, i)
            start_store(p, i * CHUNK)
        return carry

    lax.fori_loop(0, N_CHUNKS // NSETS, pipe_body, 0, unroll=False)

    for s in range(1, NSETS):
        wait_store(s)


def _tc_kernel(x_ref, ts_ref, vol_ref, pe_ref, ct_ref, out_ref):
    j = pl.program_id(0)
    ts = ts_ref[0, j, :].reshape(1, TC_BLK)
    hh = (ts // 3600) % 24
    dd = (ts // 86400) % 7
    mm = (ts // 2592000) % 12
    k = lax.broadcasted_iota(jnp.int32, (K_TAB, TC_BLK), 0)
    onehot = ((k == hh) | (k == (24 + dd)) | (k == (31 + mm))
              | (k == 43)).astype(jnp.float32)
    vol = vol_ref[0, j, :].reshape(1, TC_BLK)
    at = jnp.where(k == 44, vol, onehot)
    enc = lax.dot_general(at, ct_ref[...], (((0,), (0,)), ((), ())),
                          preferred_element_type=jnp.float32)
    out_ref[0, :, :] = x_ref[0, :, :] + pe_ref[...] + enc


def _run_tc(x, ts, vol2d, pe_s, ctable):
    grid = (S_TC // TC_BLK, B)
    ts3 = ts.reshape(B, S // TC_BLK, TC_BLK)
    vol3 = vol2d.reshape(B, S // TC_BLK, TC_BLK)
    return pl.pallas_call(
        _tc_kernel,
        grid=grid,
        in_specs=[
            pl.BlockSpec((1, TC_BLK, D_MODEL), lambda j, b: (b, j, 0)),
            pl.BlockSpec((1, S // TC_BLK, TC_BLK), lambda j, b: (b, 0, 0)),
            pl.BlockSpec((1, S // TC_BLK, TC_BLK), lambda j, b: (b, 0, 0)),
            pl.BlockSpec((TC_BLK, D_MODEL), lambda j, b: (j, 0)),
            pl.BlockSpec((K_TAB, D_MODEL), lambda j, b: (0, 0)),
        ],
        out_specs=pl.BlockSpec((1, TC_BLK, D_MODEL), lambda j, b: (b, j, 0)),
        out_shape=jax.ShapeDtypeStruct((B, S, D_MODEL), jnp.float32),
    )(x, ts3, vol3, pe_s, ctable)


def kernel(x, timestamps, volatility_regime, pe, hour_table, day_table,
           month_table, regime_W, regime_b):
    vol2d = volatility_regime[..., 0]
    wb = jnp.stack([regime_W[:, 0], regime_b])
    pe_s = pe
    consts = jnp.tile(
        jnp.array([3600, 24, 7, 30, 12], jnp.int32)[:, None], (1, L))
    ct = jnp.zeros((K_TAB, D_MODEL), jnp.float32)
    ct = ct.at[0:24, 0:D4].set(hour_table)
    ct = ct.at[24:31, D4:2 * D4].set(day_table)
    ct = ct.at[31:43, 2 * D4:3 * D4].set(month_table)
    ct = ct.at[43, 3 * D4:].set(regime_b)
    ct = ct.at[44, 3 * D4:].set(regime_W[:, 0])

    mesh = plsc.VectorSubcoreMesh(core_axis_name="c", subcore_axis_name="s")
    scratch = []
    for _ in range(NSETS * B):
        scratch.append(pltpu.VMEM((CHUNK, D_MODEL), jnp.float32))
    for _ in range(NSETS):
        scratch.append(pltpu.VMEM((CHUNK, D_MODEL), jnp.float32))
    scratch += [
        pltpu.VMEM((24 * D4,), jnp.float32),
        pltpu.VMEM((7 * D4,), jnp.float32),
        pltpu.VMEM((12 * D4,), jnp.float32),
        pltpu.VMEM((2, D4), jnp.float32),
        pltpu.VMEM((B, S_PER_W), jnp.int32),
        pltpu.VMEM((B, S_PER_W), jnp.float32),
        pltpu.VMEM((B, S_PER_W), jnp.int32),
        pltpu.VMEM((B, S_PER_W), jnp.int32),
        pltpu.VMEM((B, S_PER_W), jnp.int32),
        pltpu.VMEM((5, L), jnp.int32),
    ]
    scratch += [pltpu.SemaphoreType.DMA] * (2 * NSETS)

    run = functools.partial(
        pl.kernel,
        out_type=jax.ShapeDtypeStruct((B, S_SC, D_MODEL), jnp.float32),
        mesh=mesh,
        compiler_params=pltpu.CompilerParams(needs_layout_passes=False),
        scratch_types=scratch,
    )(_sc_kernel)
    sc_part = run(x, timestamps.reshape(-1), vol2d.reshape(-1), pe_s,
                  hour_table.reshape(-1), day_table.reshape(-1),
                  month_table.reshape(-1), wb, consts)
    tc_out = _run_tc(x, timestamps, vol2d, pe_s, ct)
    return lax.dynamic_update_slice(tc_out, sc_part, (0, S_TC, 0))


# speedup vs baseline: 1.2405x; 1.1279x over previous
import functools

import jax
import jax.numpy as jnp
from jax import lax
from jax.experimental import pallas as pl
from jax.experimental.pallas import tpu as pltpu
from jax.experimental.pallas import tpu_sc as plsc

D_MODEL = 1024
D4 = D_MODEL // 4
B = 4
S = 4096
NC = 2
NS = 16
NW = NC * NS
S_SC = 512
S_TC = S - S_SC
S_PER_W = S_SC // NW
CHUNK = 4
N_CHUNKS = S_PER_W // CHUNK
NSETS = 4
L = 16
TC_BLK = 1024
K_TAB = 64


def _sc_kernel(x_hbm, ts_hbm, vol_hbm, pe_hbm, h_hbm, d_hbm, m_hbm, wb_hbm,
               consts_hbm, out_hbm, *scratch):
    xs = [list(scratch[s * B:(s + 1) * B]) for s in range(NSETS)]
    pebufs = list(scratch[16:20])
    htab, dtab, mtab, wb = scratch[20:24]
    tsv, volv, hbv, dbv, mbv, cbuf = scratch[24:30]
    lsems = list(scratch[30:34])
    ssems = list(scratch[34:38])

    wid = lax.axis_index("s") * NC + lax.axis_index("c")
    s0 = S_TC + wid * S_PER_W
    o0 = wid * S_PER_W

    pltpu.sync_copy(h_hbm, htab)
    pltpu.sync_copy(d_hbm, dtab)
    pltpu.sync_copy(m_hbm, mtab)
    pltpu.sync_copy(wb_hbm, wb)
    pltpu.sync_copy(consts_hbm, cbuf)
    for b in range(B):
        pltpu.sync_copy(ts_hbm.at[pl.ds(b * S + s0, S_PER_W)], tsv.at[b])
        pltpu.sync_copy(vol_hbm.at[pl.ds(b * S + s0, S_PER_W)], volv.at[b])

    def idx_body(g, carry):
        off = g * L
        sl = pl.ds(off, L)
        for b in range(B):
            tsg = tsv[b, sl].astype(jnp.uint32)
            cu = [cbuf[i, :].astype(jnp.uint32) for i in range(5)]
            q1 = lax.div(tsg, cu[0])
            days = lax.div(q1, cu[1])
            hbv[b, sl] = ((q1 - days * cu[1]).astype(jnp.int32)) << 8
            dbv[b, sl] = ((days - lax.div(days, cu[2]) * cu[2])
                          .astype(jnp.int32)) << 8
            mo = lax.div(days, cu[3])
            mbv[b, sl] = ((mo - lax.div(mo, cu[4]) * cu[4])
                          .astype(jnp.int32)) << 8
        return carry

    lax.fori_loop(0, S_PER_W // L, idx_body, 0, unroll=False)

    iot = lax.broadcasted_iota(jnp.int32, (L,), 0)

    def start_load(s, row0):
        pltpu.async_copy(pe_hbm.at[pl.ds(s0 + row0, CHUNK), :],
                         pebufs[s], lsems[s])
        for b in range(B):
            pltpu.async_copy(x_hbm.at[b, pl.ds(s0 + row0, CHUNK), :],
                             xs[s][b], lsems[s])

    def wait_load(s):
        pltpu.make_async_copy(pe_hbm.at[pl.ds(0, CHUNK), :],
                              pebufs[s], lsems[s]).wait()
        for b in range(B):
            pltpu.make_async_copy(x_hbm.at[b, pl.ds(0, CHUNK), :],
                                  xs[s][b], lsems[s]).wait()

    def start_store(s, row0):
        for b in range(B):
            pltpu.async_copy(xs[s][b],
                             out_hbm.at[b, pl.ds(o0 + row0, CHUNK), :],
                             ssems[s])

    def wait_store(s):
        for b in range(B):
            pltpu.make_async_copy(xs[s][b],
                                  out_hbm.at[b, pl.ds(0, CHUNK), :],
                                  ssems[s]).wait()

    def compute(s, c):
        pebuf = pebufs[s]
        xbufs = xs[s]

        def token_body(t, carry):
            col = c * CHUNK + t
            hv = [None] * B
            dv = [None] * B
            mv = [None] * B
            vv = [None] * B
            tvec = jnp.full((L,), t, dtype=jnp.int32)
            cvec = jnp.full((L,), col, dtype=jnp.int32)
            for b in range(B):
                bvec = jnp.full((L,), b, dtype=jnp.int32)
                hv[b] = plsc.load_gather(hbv, [bvec, cvec])
                dv[b] = plsc.load_gather(dbv, [bvec, cvec])
                mv[b] = plsc.load_gather(mbv, [bvec, cvec])
                vv[b] = plsc.load_gather(volv, [bvec, cvec])
            for j in range(16):
                colj = iot + (j * L)
                col1 = colj + D4
                col2 = colj + 2 * D4
                col3 = colj + 3 * D4
                c0 = pl.ds(j * L, L)
                c1 = pl.ds(D4 + j * L, L)
                c2 = pl.ds(2 * D4 + j * L, L)
                c3 = pl.ds(3 * D4 + j * L, L)
                pe0 = pebuf[t, c0]
                pe1 = pebuf[t, c1]
                pe2 = pebuf[t, c2]
                pe3 = pebuf[t, c3]
                wW = wb[0, c0]
                wB = wb[1, c0]
                for b in range(B):
                    xb = xbufs[b]
                    hrow = plsc.load_gather(htab, [hv[b] + colj])
                    drow = plsc.load_gather(dtab, [dv[b] + colj])
                    mrow = plsc.load_gather(mtab, [mv[b] + colj])
                    plsc.addupdate_scatter(xb, [tvec, colj], pe0 + hrow)
                    plsc.addupdate_scatter(xb, [tvec, col1], pe1 + drow)
                    plsc.addupdate_scatter(xb, [tvec, col2], pe2 + mrow)
                    plsc.addupdate_scatter(xb, [tvec, col3],
                                           pe3 + (wW * vv[b] + wB))
            return carry

        lax.fori_loop(0, CHUNK, token_body, 0, unroll=False)

    start_load(0, 0)

    def pipe_body(k, carry):
        for p in range(NSETS):
            i = k * NSETS + p
            s_next = (p + 1) % NSETS

            @pl.when(i >= NSETS - 1)
            def _():
                wait_store(s_next)

            @pl.when(i + 1 < N_CHUNKS)
            def _():
                start_load(s_next, (i + 1) * CHUNK)

            wait_load(p)
            compute(p, i)
            start_store(p, i * CHUNK)
        return carry

    lax.fori_loop(0, N_CHUNKS // NSETS, pipe_body, 0, unroll=False)

    for s in range(1, NSETS):
        wait_store(s)


def _tc_kernel(x_ref, ts_ref, vol_ref, pe_ref, ct_ref, out_ref):
    j = pl.program_id(0)
    ts = ts_ref[0, j, :].reshape(1, TC_BLK)
    hh = (ts // 3600) % 24
    dd = (ts // 86400) % 7
    mm = (ts // 2592000) % 12
    k = lax.broadcasted_iota(jnp.int32, (K_TAB, TC_BLK), 0)
    onehot = ((k == hh) | (k == (24 + dd)) | (k == (31 + mm))
              | (k == 43)).astype(jnp.float32)
    vol = vol_ref[0, j, :].reshape(1, TC_BLK)
    at = jnp.where(k == 44, vol, onehot)
    enc = lax.dot_general(at, ct_ref[...], (((0,), (0,)), ((), ())),
                          preferred_element_type=jnp.float32)
    out_ref[0, :, :] = x_ref[0, :, :] + pe_ref[...] + enc


def _run_tc(x, ts, vol2d, pe_s, ctable):
    grid = (S_TC // TC_BLK, B)
    ts3 = ts.reshape(B, S // TC_BLK, TC_BLK)
    vol3 = vol2d.reshape(B, S // TC_BLK, TC_BLK)
    return pl.pallas_call(
        _tc_kernel,
        grid=grid,
        in_specs=[
            pl.BlockSpec((1, TC_BLK, D_MODEL), lambda j, b: (b, j, 0)),
            pl.BlockSpec((1, S // TC_BLK, TC_BLK), lambda j, b: (b, 0, 0)),
            pl.BlockSpec((1, S // TC_BLK, TC_BLK), lambda j, b: (b, 0, 0)),
            pl.BlockSpec((TC_BLK, D_MODEL), lambda j, b: (j, 0)),
            pl.BlockSpec((K_TAB, D_MODEL), lambda j, b: (0, 0)),
        ],
        out_specs=pl.BlockSpec((1, TC_BLK, D_MODEL), lambda j, b: (b, j, 0)),
        out_shape=jax.ShapeDtypeStruct((B, S, D_MODEL), jnp.float32),
    )(x, ts3, vol3, pe_s, ctable)


def kernel(x, timestamps, volatility_regime, pe, hour_table, day_table,
           month_table, regime_W, regime_b):
    vol2d = volatility_regime[..., 0]
    wb = jnp.stack([regime_W[:, 0], regime_b])
    pe_s = pe
    consts = jnp.tile(
        jnp.array([3600, 24, 7, 30, 12], jnp.int32)[:, None], (1, L))
    ct = jnp.zeros((K_TAB, D_MODEL), jnp.float32)
    ct = ct.at[0:24, 0:D4].set(hour_table)
    ct = ct.at[24:31, D4:2 * D4].set(day_table)
    ct = ct.at[31:43, 2 * D4:3 * D4].set(month_table)
    ct = ct.at[43, 3 * D4:].set(regime_b)
    ct = ct.at[44, 3 * D4:].set(regime_W[:, 0])

    mesh = plsc.VectorSubcoreMesh(core_axis_name="c", subcore_axis_name="s")
    scratch = []
    for _ in range(NSETS * B):
        scratch.append(pltpu.VMEM((CHUNK, D_MODEL), jnp.float32))
    for _ in range(NSETS):
        scratch.append(pltpu.VMEM((CHUNK, D_MODEL), jnp.float32))
    scratch += [
        pltpu.VMEM((24 * D4,), jnp.float32),
        pltpu.VMEM((7 * D4,), jnp.float32),
        pltpu.VMEM((12 * D4,), jnp.float32),
        pltpu.VMEM((2, D4), jnp.float32),
        pltpu.VMEM((B, S_PER_W), jnp.int32),
        pltpu.VMEM((B, S_PER_W), jnp.float32),
        pltpu.VMEM((B, S_PER_W), jnp.int32),
        pltpu.VMEM((B, S_PER_W), jnp.int32),
        pltpu.VMEM((B, S_PER_W), jnp.int32),
        pltpu.VMEM((5, L), jnp.int32),
    ]
    scratch += [pltpu.SemaphoreType.DMA] * (2 * NSETS)

    run = functools.partial(
        pl.kernel,
        out_type=jax.ShapeDtypeStruct((B, S_SC, D_MODEL), jnp.float32),
        mesh=mesh,
        compiler_params=pltpu.CompilerParams(needs_layout_passes=False),
        scratch_types=scratch,
    )(_sc_kernel)
    sc_part = run(x, timestamps.reshape(-1), vol2d.reshape(-1), pe_s,
                  hour_table.reshape(-1), day_table.reshape(-1),
                  month_table.reshape(-1), wb, consts)
    tc_out = _run_tc(x, timestamps, vol2d, pe_s, ct)
    return lax.dynamic_update_slice(tc_out, sc_part, (0, S_TC, 0))
